# baseline (speedup 1.0000x reference)
import functools

import jax
import jax.numpy as jnp
import numpy as np
from jax import lax
from jax.experimental import pallas as pl
from jax.experimental.pallas import tpu as pltpu
from jax.experimental.pallas import tpu_sc as plsc

_B = 4096
_F = 26
_D = 16
_NPAIR = (_F * (_F - 1)) // 2
_EPS = 1e-5

_NC = 2
_NS = 16
_NW = _NC * _NS
_ROWS = _B * _F
_RPW = _ROWS // _NW
_CHUNK = 128
_NCHUNK = _RPW // _CHUNK

_II, _JJ = np.triu_indices(_F, k=1)


def _sc_gather(idx_flat, emb_table, coeff_flat):
  mesh = plsc.VectorSubcoreMesh(core_axis_name="c", subcore_axis_name="s")

  @functools.partial(
      pl.kernel,
      mesh=mesh,
      compiler_params=pltpu.CompilerParams(use_tc_tiling_on_sc=False),
      out_type=(
          jax.ShapeDtypeStruct((_ROWS // _CHUNK, _CHUNK, _D), jnp.float32),
          jax.ShapeDtypeStruct((_ROWS,), jnp.float32),
      ),
      scratch_types=[
          pltpu.VMEM((_RPW,), jnp.int32),
          pltpu.VMEM((_NCHUNK, _CHUNK, _D), jnp.float32),
          pltpu.VMEM((_RPW,), jnp.float32),
          pltpu.SemaphoreType.DMA,
          pltpu.SemaphoreType.DMA,
      ],
  )
  def gather_kernel(idx_hbm, table_hbm, coeff_hbm, emb_out, coeff_out,
                    idx_v, rows_v, coeff_v, sem_e, sem_c):
    wid = lax.axis_index("s") * _NC + lax.axis_index("c")
    pltpu.sync_copy(idx_hbm.at[pl.ds(wid * _RPW, _RPW)], idx_v)
    copies = []
    for j in range(_NCHUNK):
      idx_j = idx_v.at[pl.ds(j * _CHUNK, _CHUNK)]
      copies.append(
          pltpu.async_copy(table_hbm.at[idx_j], rows_v.at[j], sem_e))
      copies.append(
          pltpu.async_copy(coeff_hbm.at[idx_j],
                           coeff_v.at[pl.ds(j * _CHUNK, _CHUNK)], sem_c))
    for cp in copies:
      cp.wait()
    pltpu.sync_copy(rows_v, emb_out.at[pl.ds(wid * _NCHUNK, _NCHUNK)])
    pltpu.sync_copy(coeff_v, coeff_out.at[pl.ds(wid * _RPW, _RPW)])

  return gather_kernel(idx_flat, emb_table, coeff_flat)


def _tc_body(sc_ref, ii_ref, jj_ref, emb_ref, coeff_ref, out_ref, squ_s, inv_s):
  e = emb_ref[...]
  squ = jnp.sum(e * e, axis=1)
  squ_c = jnp.clip(squ, 0.0, 1.0 - _EPS)
  squ_s[...] = squ_c
  inv_s[...] = 1.0 / (1.0 - squ_c)

  bb = out_ref.shape[0]

  def pair_body(p, acc):
    i = ii_ref[p]
    j = jj_ref[p]
    xi = emb_ref[i]
    xj = emb_ref[j]
    dot = jnp.sum(xi * xj, axis=0)
    sqd = squ_s[i] + squ_s[j] - 2.0 * dot
    x = 2.0 * sqd * (inv_s[i] * inv_s[j]) + 1.0
    x = jnp.maximum(x, 1.0)
    z = jnp.sqrt(x * x - 1.0)
    return acc + jnp.log(x + z)

  acc = lax.fori_loop(0, _NPAIR, pair_body, jnp.zeros((bb,), jnp.float32))
  beta = sc_ref[0]
  c = sc_ref[1]
  bias = sc_ref[2]
  lin = jnp.sum(coeff_ref[...], axis=0)
  out_ref[...] = beta * acc + (float(_NPAIR) * c + bias) + lin


def _tc_compute(emb_t, coeff_t, scalars, interpret=False):
  bb = 1024
  grid = _B // bb
  ii = jnp.asarray(_II, dtype=jnp.int32)
  jj = jnp.asarray(_JJ, dtype=jnp.int32)
  return pl.pallas_call(
      _tc_body,
      grid=(grid,),
      in_specs=[
          pl.BlockSpec(memory_space=pltpu.SMEM),
          pl.BlockSpec(memory_space=pltpu.SMEM),
          pl.BlockSpec(memory_space=pltpu.SMEM),
          pl.BlockSpec((_F, _D, bb), lambda b: (0, 0, b)),
          pl.BlockSpec((_F, bb), lambda b: (0, b)),
      ],
      out_specs=pl.BlockSpec((bb,), lambda b: (b,)),
      out_shape=jax.ShapeDtypeStruct((_B,), jnp.float32),
      scratch_shapes=[
          pltpu.VMEM((_F, bb), jnp.float32),
          pltpu.VMEM((_F, bb), jnp.float32),
      ],
      interpret=interpret,
  )(scalars, ii, jj, emb_t, coeff_t)


def kernel(features, emb_table, coeff_table, bias, beta, c):
  idx_flat = features.astype(jnp.int32).reshape(_ROWS)
  coeff_flat = coeff_table.reshape(-1)

  emb_rows, coeff_rows = _sc_gather(idx_flat, emb_table, coeff_flat)

  emb_t = emb_rows.reshape(_B, _F, _D).transpose(1, 2, 0)
  coeff_t = coeff_rows.reshape(_B, _F).transpose(1, 0)
  scalars = jnp.concatenate([beta, c, bias]).astype(jnp.float32)

  return _tc_compute(emb_t, coeff_t, scalars)


# speedup vs baseline: 1.1286x; 1.1286x over previous
import functools

import jax
import jax.numpy as jnp
import numpy as np
from jax import lax
from jax.experimental import pallas as pl
from jax.experimental.pallas import tpu as pltpu
from jax.experimental.pallas import tpu_sc as plsc

_B = 4096
_F = 26
_D = 16
_NPAIR = (_F * (_F - 1)) // 2
_EPS = 1e-5

_NC = 2
_HALF = _B // 2
_CHUNK = 128
_NCH = _HALF // _CHUNK

_II, _JJ = np.triu_indices(_F, k=1)


def _sc_gather(idx_fmajor, emb_table, coeff_flat):
  mesh = plsc.VectorSubcoreMesh(core_axis_name="c", subcore_axis_name="s")

  @functools.partial(
      pl.kernel,
      mesh=mesh,
      compiler_params=pltpu.CompilerParams(
          use_tc_tiling_on_sc=False, needs_layout_passes=False),
      out_type=(
          jax.ShapeDtypeStruct((_F, _D, _B), jnp.float32),
          jax.ShapeDtypeStruct((_F, _B), jnp.float32),
      ),
      scratch_types=[
          pltpu.VMEM((_B,), jnp.int32),
          pltpu.VMEM((_HALF, _D), jnp.float32),
          pltpu.VMEM((_HALF, _D), jnp.float32),
          pltpu.VMEM((_D * _HALF,), jnp.float32),
          pltpu.VMEM((_B,), jnp.float32),
          pltpu.SemaphoreType.DMA,
          pltpu.SemaphoreType.DMA,
          pltpu.SemaphoreType.DMA,
          pltpu.SemaphoreType.DMA,
      ],
  )
  def gather_kernel(idx_hbm, table_hbm, coeff_hbm, emb_out, coeff_out,
                    idx_v, rows_a, rows_b, tr_v, coeff_v,
                    sem_a, sem_b, sem_c, sem_w):
    f = lax.axis_index("s") * _NC + lax.axis_index("c")

    lanes = lax.iota(jnp.int32, _D)

    def transpose_half(rows_ref):
      def body(k, _):
        base = k * _D
        for r in range(_D):
          vec = rows_ref[base + r]
          idx = lanes * _HALF + (base + r)
          plsc.store_scatter(tr_v, [idx], vec)
        return 0
      lax.fori_loop(0, _HALF // _D, body, 0)

    @pl.when(f < _F)
    def _():
      pltpu.sync_copy(idx_hbm.at[pl.ds(f * _B, _B)], idx_v)
      cps_a = [
          pltpu.async_copy(
              table_hbm.at[idx_v.at[pl.ds(c * _CHUNK, _CHUNK)]],
              rows_a.at[pl.ds(c * _CHUNK, _CHUNK)], sem_a)
          for c in range(_NCH)
      ]
      cps_c = [
          pltpu.async_copy(
              coeff_hbm.at[idx_v.at[pl.ds(c * _CHUNK, _CHUNK)]],
              coeff_v.at[pl.ds(c * _CHUNK, _CHUNK)], sem_c)
          for c in range(2 * _NCH)
      ]
      cps_b = [
          pltpu.async_copy(
              table_hbm.at[idx_v.at[pl.ds(_HALF + c * _CHUNK, _CHUNK)]],
              rows_b.at[pl.ds(c * _CHUNK, _CHUNK)], sem_b)
          for c in range(_NCH)
      ]
      for cp in cps_a:
        cp.wait()
      transpose_half(rows_a)
      wr0 = [
          pltpu.async_copy(tr_v.at[pl.ds(d * _HALF, _HALF)],
                           emb_out.at[f, d, pl.ds(0, _HALF)], sem_w)
          for d in range(_D)
      ]
      for cp in cps_b:
        cp.wait()
      for cp in wr0:
        cp.wait()
      transpose_half(rows_b)
      wr1 = [
          pltpu.async_copy(tr_v.at[pl.ds(d * _HALF, _HALF)],
                           emb_out.at[f, d, pl.ds(_HALF, _HALF)], sem_w)
          for d in range(_D)
      ]
      for cp in cps_c:
        cp.wait()
      pltpu.sync_copy(coeff_v, coeff_out.at[f])
      for cp in wr1:
        cp.wait()

  return gather_kernel(idx_fmajor, emb_table, coeff_flat)


def _tc_body(sc_ref, emb_ref, coeff_ref, out_ref):
  e = emb_ref[...]
  squ = jnp.sum(e * e, axis=1)
  squ_c = jnp.clip(squ, 0.0, 1.0 - _EPS)
  inv = 1.0 / (1.0 - squ_c)

  n_acc = 8
  accs = [jnp.zeros((out_ref.shape[0],), jnp.float32) for _ in range(n_acc)]
  for p in range(_NPAIR):
    i = int(_II[p])
    j = int(_JJ[p])
    dot = jnp.sum(e[i] * e[j], axis=0)
    sqd = squ_c[i] + squ_c[j] - 2.0 * dot
    x = 2.0 * sqd * (inv[i] * inv[j]) + 1.0
    x = jnp.maximum(x, 1.0)
    z = jnp.sqrt(x * x - 1.0)
    accs[p % n_acc] = accs[p % n_acc] + jnp.log(x + z)
  acc = accs[0]
  for a in accs[1:]:
    acc = acc + a

  beta = sc_ref[0]
  c = sc_ref[1]
  bias = sc_ref[2]
  lin = jnp.sum(coeff_ref[...], axis=0)
  out_ref[...] = beta * acc + (float(_NPAIR) * c + bias) + lin


def _tc_compute(emb_t, coeff_t, scalars, interpret=False):
  bb = 1024
  grid = _B // bb
  return pl.pallas_call(
      _tc_body,
      grid=(grid,),
      in_specs=[
          pl.BlockSpec(memory_space=pltpu.SMEM),
          pl.BlockSpec((_F, _D, bb), lambda b: (0, 0, b)),
          pl.BlockSpec((_F, bb), lambda b: (0, b)),
      ],
      out_specs=pl.BlockSpec((bb,), lambda b: (b,)),
      out_shape=jax.ShapeDtypeStruct((_B,), jnp.float32),
      interpret=interpret,
  )(scalars, emb_t, coeff_t)


def kernel(features, emb_table, coeff_table, bias, beta, c):
  idx_fmajor = features.astype(jnp.int32).T.reshape(_F * _B)
  coeff_flat = coeff_table.reshape(-1)

  emb_t, coeff_t = _sc_gather(idx_fmajor, emb_table, coeff_flat)
  scalars = jnp.concatenate([beta, c, bias]).astype(jnp.float32)

  return _tc_compute(emb_t, coeff_t, scalars)


# speedup vs baseline: 1.2044x; 1.0671x over previous
import functools

import jax
import jax.numpy as jnp
import numpy as np
from jax import lax
from jax.experimental import pallas as pl
from jax.experimental.pallas import tpu as pltpu
from jax.experimental.pallas import tpu_sc as plsc

_B = 4096
_F = 26
_D = 16
_NPAIR = (_F * (_F - 1)) // 2
_EPS = 1e-5

_NC = 2
_HALF = _B // 2
_CHUNK = 128
_NCH = _HALF // _CHUNK

_II, _JJ = np.triu_indices(_F, k=1)


def _sc_gather(idx_fmajor, emb_table, coeff_flat):
  mesh = plsc.VectorSubcoreMesh(core_axis_name="c", subcore_axis_name="s")

  @functools.partial(
      pl.kernel,
      mesh=mesh,
      compiler_params=pltpu.CompilerParams(
          use_tc_tiling_on_sc=False, needs_layout_passes=False),
      out_type=(
          jax.ShapeDtypeStruct((_F, _D, _B), jnp.float32),
          jax.ShapeDtypeStruct((_F, _B), jnp.float32),
      ),
      scratch_types=[
          pltpu.VMEM((_B,), jnp.int32),
          pltpu.VMEM((_B,), jnp.int32),
          pltpu.VMEM((_HALF, _D), jnp.float32),
          pltpu.VMEM((_HALF, _D), jnp.float32),
          pltpu.VMEM((_D * _HALF,), jnp.float32),
          pltpu.VMEM((_B,), jnp.float32),
          pltpu.SemaphoreType.DMA,
          pltpu.SemaphoreType.DMA,
          pltpu.SemaphoreType.DMA,
          pltpu.SemaphoreType.DMA,
          pltpu.SemaphoreType.DMA,
      ],
  )
  def gather_kernel(feat_hbm, table_hbm, coeff_hbm, emb_out, coeff_out,
                    meta_v, idx_v, rows_a, rows_b, tr_v, coeff_v,
                    sem_m, sem_a, sem_b, sem_c, sem_w):
    f = lax.axis_index("s") * _NC + lax.axis_index("c")

    lanes = lax.iota(jnp.int32, _D)

    def transpose_half(rows_ref):
      def body(k, _):
        base = k * _D
        for r in range(_D):
          vec = rows_ref[base + r]
          idx = lanes * _HALF + (base + r)
          plsc.store_scatter(tr_v, [idx], vec)
        return 0
      lax.fori_loop(0, _HALF // _D, body, 0)

    @pl.when(f < _F)
    def _():
      def mbody(k, _):
        meta_v[pl.ds(k * _D, _D)] = (k * _D + lanes) * _F + f
        return 0
      lax.fori_loop(0, _B // _D, mbody, 0)
      cps_i = [
          pltpu.async_copy(
              feat_hbm.at[meta_v.at[pl.ds(c * _CHUNK, _CHUNK)]],
              idx_v.at[pl.ds(c * _CHUNK, _CHUNK)], sem_m)
          for c in range(_B // _CHUNK)
      ]
      for cp in cps_i:
        cp.wait()
      cps_a = [
          pltpu.async_copy(
              table_hbm.at[idx_v.at[pl.ds(c * _CHUNK, _CHUNK)]],
              rows_a.at[pl.ds(c * _CHUNK, _CHUNK)], sem_a)
          for c in range(_NCH)
      ]
      cps_c = [
          pltpu.async_copy(
              coeff_hbm.at[idx_v.at[pl.ds(c * _CHUNK, _CHUNK)]],
              coeff_v.at[pl.ds(c * _CHUNK, _CHUNK)], sem_c)
          for c in range(2 * _NCH)
      ]
      cps_b = [
          pltpu.async_copy(
              table_hbm.at[idx_v.at[pl.ds(_HALF + c * _CHUNK, _CHUNK)]],
              rows_b.at[pl.ds(c * _CHUNK, _CHUNK)], sem_b)
          for c in range(_NCH)
      ]
      for cp in cps_a:
        cp.wait()
      transpose_half(rows_a)
      wr0 = [
          pltpu.async_copy(tr_v.at[pl.ds(d * _HALF, _HALF)],
                           emb_out.at[f, d, pl.ds(0, _HALF)], sem_w)
          for d in range(_D)
      ]
      for cp in cps_b:
        cp.wait()
      for cp in wr0:
        cp.wait()
      transpose_half(rows_b)
      wr1 = [
          pltpu.async_copy(tr_v.at[pl.ds(d * _HALF, _HALF)],
                           emb_out.at[f, d, pl.ds(_HALF, _HALF)], sem_w)
          for d in range(_D)
      ]
      for cp in cps_c:
        cp.wait()
      pltpu.sync_copy(coeff_v, coeff_out.at[f])
      for cp in wr1:
        cp.wait()

  return gather_kernel(idx_fmajor, emb_table, coeff_flat)


def _tc_body(sc_ref, emb_ref, coeff_ref, out_ref):
  e = emb_ref[...]
  squ = jnp.sum(e * e, axis=1)
  squ_c = jnp.clip(squ, 0.0, 1.0 - _EPS)
  inv = 1.0 / (1.0 - squ_c)

  n_acc = 8
  accs = [jnp.zeros((8, 128), jnp.float32) for _ in range(n_acc)]
  for p in range(_NPAIR):
    i = int(_II[p])
    j = int(_JJ[p])
    dot = jnp.sum(e[i] * e[j], axis=0)
    sqd = squ_c[i] + squ_c[j] - 2.0 * dot
    x = 2.0 * sqd * (inv[i] * inv[j]) + 1.0
    x = jnp.maximum(x, 1.0)
    z = jnp.sqrt(x * x - 1.0)
    accs[p % n_acc] = accs[p % n_acc] + jnp.log(x + z)
  acc = accs[0]
  for a in accs[1:]:
    acc = acc + a

  beta = sc_ref[0]
  c = sc_ref[1]
  bias = sc_ref[2]
  lin = jnp.sum(coeff_ref[...], axis=0)
  out_ref[...] = beta * acc + (float(_NPAIR) * c + bias) + lin


def _tc_compute(emb_t, coeff_t, scalars, interpret=False):
  grid = _B // 1024
  emb4 = emb_t.reshape(_F, _D, _B // 128, 128)
  coeff3 = coeff_t.reshape(_F, _B // 128, 128)
  out = pl.pallas_call(
      _tc_body,
      grid=(grid,),
      in_specs=[
          pl.BlockSpec(memory_space=pltpu.SMEM),
          pl.BlockSpec((_F, _D, 8, 128), lambda b: (0, 0, b, 0)),
          pl.BlockSpec((_F, 8, 128), lambda b: (0, b, 0)),
      ],
      out_specs=pl.BlockSpec((8, 128), lambda b: (b, 0)),
      out_shape=jax.ShapeDtypeStruct((_B // 128, 128), jnp.float32),
      interpret=interpret,
  )(scalars, emb4, coeff3)
  return out.reshape(_B)


def kernel(features, emb_table, coeff_table, bias, beta, c):
  feat_flat = features.astype(jnp.int32).reshape(_F * _B)
  coeff_flat = coeff_table.reshape(-1)

  emb_t, coeff_t = _sc_gather(feat_flat, emb_table, coeff_flat)
  scalars = jnp.concatenate([beta, c, bias]).astype(jnp.float32)

  return _tc_compute(emb_t, coeff_t, scalars)


# speedup vs baseline: 1.2059x; 1.0012x over previous
import functools

import jax
import jax.numpy as jnp
import numpy as np
from jax import lax
from jax.experimental import pallas as pl
from jax.experimental.pallas import tpu as pltpu
from jax.experimental.pallas import tpu_sc as plsc

_B = 4096
_F = 26
_D = 16
_NPAIR = (_F * (_F - 1)) // 2
_EPS = 1e-5

_NC = 2
_HALF = _B // 2
_CHUNK = 128
_NCH = _HALF // _CHUNK

_II, _JJ = np.triu_indices(_F, k=1)


def _sc_gather(idx_fmajor, emb_table, coeff_flat):
  mesh = plsc.VectorSubcoreMesh(core_axis_name="c", subcore_axis_name="s")

  @functools.partial(
      pl.kernel,
      mesh=mesh,
      compiler_params=pltpu.CompilerParams(
          use_tc_tiling_on_sc=False, needs_layout_passes=False),
      out_type=(
          jax.ShapeDtypeStruct((_F * _D * _B,), jnp.float32),
          jax.ShapeDtypeStruct((_F * _B,), jnp.float32),
      ),
      scratch_types=[
          pltpu.VMEM((_B,), jnp.int32),
          pltpu.VMEM((_B,), jnp.int32),
          pltpu.VMEM((_HALF, _D), jnp.float32),
          pltpu.VMEM((_HALF, _D), jnp.float32),
          pltpu.VMEM((_D * _HALF,), jnp.float32),
          pltpu.VMEM((_B,), jnp.float32),
          pltpu.SemaphoreType.DMA,
          pltpu.SemaphoreType.DMA,
          pltpu.SemaphoreType.DMA,
          pltpu.SemaphoreType.DMA,
          pltpu.SemaphoreType.DMA,
      ],
  )
  def gather_kernel(feat_hbm, table_hbm, coeff_hbm, emb_out, coeff_out,
                    meta_v, idx_v, rows_a, rows_b, tr_v, coeff_v,
                    sem_m, sem_a, sem_b, sem_c, sem_w):
    f = lax.axis_index("s") * _NC + lax.axis_index("c")

    lanes = lax.iota(jnp.int32, _D)

    def transpose_half(rows_ref):
      def body(k, _):
        base = k * _D
        for r in range(_D):
          vec = rows_ref[base + r]
          idx = lanes * _HALF + (base + r)
          plsc.store_scatter(tr_v, [idx], vec)
        return 0
      lax.fori_loop(0, _HALF // _D, body, 0)

    @pl.when(f < _F)
    def _():
      def mbody(k, _):
        meta_v[pl.ds(k * _D, _D)] = (k * _D + lanes) * _F + f
        return 0
      lax.fori_loop(0, _B // _D, mbody, 0)
      cps_i = [
          pltpu.async_copy(
              feat_hbm.at[meta_v.at[pl.ds(c * _CHUNK, _CHUNK)]],
              idx_v.at[pl.ds(c * _CHUNK, _CHUNK)], sem_m)
          for c in range(_B // _CHUNK)
      ]
      for cp in cps_i:
        cp.wait()
      cps_a = [
          pltpu.async_copy(
              table_hbm.at[idx_v.at[pl.ds(c * _CHUNK, _CHUNK)]],
              rows_a.at[pl.ds(c * _CHUNK, _CHUNK)], sem_a)
          for c in range(_NCH)
      ]
      cps_c = [
          pltpu.async_copy(
              coeff_hbm.at[idx_v.at[pl.ds(c * _CHUNK, _CHUNK)]],
              coeff_v.at[pl.ds(c * _CHUNK, _CHUNK)], sem_c)
          for c in range(2 * _NCH)
      ]
      cps_b = [
          pltpu.async_copy(
              table_hbm.at[idx_v.at[pl.ds(_HALF + c * _CHUNK, _CHUNK)]],
              rows_b.at[pl.ds(c * _CHUNK, _CHUNK)], sem_b)
          for c in range(_NCH)
      ]
      for cp in cps_a:
        cp.wait()
      transpose_half(rows_a)
      wr0 = [
          pltpu.async_copy(
              tr_v.at[pl.ds(d * _HALF, _HALF)],
              emb_out.at[pl.ds(f * (_D * _B) + d * _B, _HALF)], sem_w)
          for d in range(_D)
      ]
      for cp in cps_b:
        cp.wait()
      for cp in wr0:
        cp.wait()
      transpose_half(rows_b)
      wr1 = [
          pltpu.async_copy(
              tr_v.at[pl.ds(d * _HALF, _HALF)],
              emb_out.at[pl.ds(f * (_D * _B) + d * _B + _HALF, _HALF)], sem_w)
          for d in range(_D)
      ]
      for cp in cps_c:
        cp.wait()
      pltpu.sync_copy(coeff_v, coeff_out.at[pl.ds(f * _B, _B)])
      for cp in wr1:
        cp.wait()

  return gather_kernel(idx_fmajor, emb_table, coeff_flat)


def _tc_body(sc_ref, emb_ref, coeff_ref, out_ref):
  e = emb_ref[...]
  squ = jnp.sum(e * e, axis=1)
  squ_c = jnp.clip(squ, 0.0, 1.0 - _EPS)
  inv = 1.0 / (1.0 - squ_c)

  n_acc = 8
  accs = [jnp.zeros((8, 128), jnp.float32) for _ in range(n_acc)]
  for p in range(_NPAIR):
    i = int(_II[p])
    j = int(_JJ[p])
    dot = jnp.sum(e[i] * e[j], axis=0)
    sqd = squ_c[i] + squ_c[j] - 2.0 * dot
    x = 2.0 * sqd * (inv[i] * inv[j]) + 1.0
    x = jnp.maximum(x, 1.0)
    z = jnp.sqrt(x * x - 1.0)
    accs[p % n_acc] = accs[p % n_acc] + jnp.log(x + z)
  acc = accs[0]
  for a in accs[1:]:
    acc = acc + a

  beta = sc_ref[0]
  c = sc_ref[1]
  bias = sc_ref[2]
  lin = jnp.sum(coeff_ref[...], axis=0)
  out_ref[...] = beta * acc + (float(_NPAIR) * c + bias) + lin


def _tc_compute(emb_t, coeff_t, scalars, interpret=False):
  grid = _B // 1024
  emb4 = emb_t.reshape(_F, _D, _B // 128, 128)
  coeff3 = coeff_t.reshape(_F, _B // 128, 128)
  out = pl.pallas_call(
      _tc_body,
      grid=(grid,),
      in_specs=[
          pl.BlockSpec(memory_space=pltpu.SMEM),
          pl.BlockSpec((_F, _D, 8, 128), lambda b: (0, 0, b, 0)),
          pl.BlockSpec((_F, 8, 128), lambda b: (0, b, 0)),
      ],
      out_specs=pl.BlockSpec((8, 128), lambda b: (b, 0)),
      out_shape=jax.ShapeDtypeStruct((_B // 128, 128), jnp.float32),
      interpret=interpret,
  )(scalars, emb4, coeff3)
  return out.reshape(_B)


def kernel(features, emb_table, coeff_table, bias, beta, c):
  feat_flat = features.astype(jnp.int32).reshape(_F * _B)
  coeff_flat = coeff_table.reshape(-1)

  emb_t, coeff_t = _sc_gather(feat_flat, emb_table, coeff_flat)
  scalars = jnp.concatenate([beta, c, bias]).astype(jnp.float32)

  return _tc_compute(emb_t, coeff_t, scalars)
